# Initial kernel scaffold; baseline (speedup 1.0000x reference)
import jax
import jax.numpy as jnp
from jax.experimental import pallas as pl


def kernel(x, adj, edge_index, W0, b0, pre1W, pre1b, post1W, post1b, lin1W, lin1b, pre2W, pre2b, post2W, post2b, lin2W, lin2b):
    raise NotImplementedError("write your pallas kernel here")


# speedup vs baseline: 1.4915x; 1.4915x over previous
import functools
import math

import jax
import jax.numpy as jnp
from jax.experimental import pallas as pl

N_NODES = 10000
NFEAT = 128
ALPHA = 0.2
AVG_DEG_LOG = math.log(33.0)
BN = 1000


def _leaky(v):
    return jnp.where(v >= 0, v, ALPHA * v)


def _full_spec(shape):
    return pl.BlockSpec(shape, lambda i: (0,) * len(shape))


def _row_spec(cols):
    return pl.BlockSpec((BN, cols), lambda i: (i, 0))


def _stage_a_body(x_ref, w0t_ref, b0_ref, p1dt_ref, p1st_ref, pre1b_ref,
                  h_ref, a_ref, b_ref):
    h = _leaky(jnp.dot(x_ref[...], w0t_ref[...],
                       preferred_element_type=jnp.float32) + b0_ref[...])
    h_ref[...] = h
    a_ref[...] = jnp.dot(h, p1dt_ref[...],
                         preferred_element_type=jnp.float32) + pre1b_ref[...]
    b_ref[...] = jnp.dot(h, p1st_ref[...], preferred_element_type=jnp.float32)


def _stage_b_body(h_ref, a1_ref, smax_ref, deg_ref,
                  pht_ref, pat_ref, pgt_ref, postb_ref, lint_ref, linb_ref,
                  p2dt_ref, p2st_ref, pre2b_ref,
                  h2_ref, a2_ref, b2_ref):
    deg = deg_ref[...]
    agg = jnp.where(deg > 0, a1_ref[...] + smax_ref[...], 0.0)
    s = jnp.log(jnp.maximum(deg, 1.0) + 1.0) * (1.0 / AVG_DEG_LOG)
    amp = agg * s
    y = (jnp.dot(h_ref[...], pht_ref[...], preferred_element_type=jnp.float32)
         + jnp.dot(amp, pat_ref[...], preferred_element_type=jnp.float32)
         + jnp.dot(agg, pgt_ref[...], preferred_element_type=jnp.float32)
         + postb_ref[...])
    y = jnp.dot(y, lint_ref[...], preferred_element_type=jnp.float32) + linb_ref[...]
    h2 = _leaky(y)
    h2_ref[...] = h2
    a2_ref[...] = jnp.dot(h2, p2dt_ref[...],
                          preferred_element_type=jnp.float32) + pre2b_ref[...]
    b2_ref[...] = jnp.dot(h2, p2st_ref[...], preferred_element_type=jnp.float32)


def _stage_c_body(h2_ref, a2_ref, smax_ref, deg_ref,
                  pht_ref, pat_ref, pgt_ref, postb_ref, lint_ref, linb_ref,
                  out_ref):
    deg = deg_ref[...]
    agg = jnp.where(deg > 0, a2_ref[...] + smax_ref[...], 0.0)
    s = jnp.log(jnp.maximum(deg, 1.0) + 1.0) * (1.0 / AVG_DEG_LOG)
    amp = agg * s
    z = (jnp.dot(h2_ref[...], pht_ref[...], preferred_element_type=jnp.float32)
         + jnp.dot(amp, pat_ref[...], preferred_element_type=jnp.float32)
         + jnp.dot(agg, pgt_ref[...], preferred_element_type=jnp.float32)
         + postb_ref[...])
    z = jnp.dot(z, lint_ref[...], preferred_element_type=jnp.float32) + linb_ref[...]
    m = jnp.max(z, axis=1, keepdims=True)
    lse = jnp.log(jnp.sum(jnp.exp(z - m), axis=1, keepdims=True)) + m
    out_ref[...] = z - lse


def _dense_a(x, w0t, b0, p1dt, p1st, pre1b):
    n = x.shape[0]
    f = jnp.float32
    return pl.pallas_call(
        _stage_a_body,
        grid=(n // BN,),
        in_specs=[_row_spec(NFEAT), _full_spec(w0t.shape), _full_spec(b0.shape),
                  _full_spec(p1dt.shape), _full_spec(p1st.shape),
                  _full_spec(pre1b.shape)],
        out_specs=[_row_spec(NFEAT)] * 3,
        out_shape=[jax.ShapeDtypeStruct((n, NFEAT), f)] * 3,
    )(x, w0t, b0, p1dt, p1st, pre1b)


def _dense_b(h, a1, smax, deg, pht, pat, pgt, postb, lint, linb,
             p2dt, p2st, pre2b):
    n = h.shape[0]
    f = jnp.float32
    return pl.pallas_call(
        _stage_b_body,
        grid=(n // BN,),
        in_specs=[_row_spec(NFEAT), _row_spec(NFEAT), _row_spec(NFEAT),
                  _row_spec(1),
                  _full_spec(pht.shape), _full_spec(pat.shape),
                  _full_spec(pgt.shape), _full_spec(postb.shape),
                  _full_spec(lint.shape), _full_spec(linb.shape),
                  _full_spec(p2dt.shape), _full_spec(p2st.shape),
                  _full_spec(pre2b.shape)],
        out_specs=[_row_spec(NFEAT)] * 3,
        out_shape=[jax.ShapeDtypeStruct((n, NFEAT), f)] * 3,
    )(h, a1, smax, deg, pht, pat, pgt, postb, lint, linb, p2dt, p2st, pre2b)


def _dense_c(h2, a2, smax, deg, pht, pat, pgt, postb, lint, linb, nclass):
    n = h2.shape[0]
    return pl.pallas_call(
        _stage_c_body,
        grid=(n // BN,),
        in_specs=[_row_spec(NFEAT), _row_spec(NFEAT), _row_spec(NFEAT),
                  _row_spec(1),
                  _full_spec(pht.shape), _full_spec(pat.shape),
                  _full_spec(pgt.shape), _full_spec(postb.shape),
                  _full_spec(lint.shape), _full_spec(linb.shape)],
        out_specs=pl.BlockSpec((BN, nclass), lambda i: (i, 0)),
        out_shape=jax.ShapeDtypeStruct((n, nclass), jnp.float32),
    )(h2, a2, smax, deg, pht, pat, pgt, postb, lint, linb)


def _segmax_deg(b, src, dst, n):
    smax = jax.ops.segment_max(b[src], dst, num_segments=n)
    return smax


def kernel(x, adj, edge_index, W0, b0, pre1W, pre1b, post1W, post1b, lin1W,
           lin1b, pre2W, pre2b, post2W, post2b, lin2W, lin2b):
    del adj
    src = edge_index[0]
    dst = edge_index[1]
    n = x.shape[0]
    f = NFEAT
    nclass = post2W.shape[0]

    w0t = W0.T
    p1dt = pre1W[:, :f].T
    p1st = pre1W[:, f:].T
    p1ht = post1W[:, :f].T
    p1at = post1W[:, f:2 * f].T
    p1gt = post1W[:, 2 * f:].T
    l1t = lin1W.T
    p2dt = pre2W[:, :f].T
    p2st = pre2W[:, f:].T
    p2ht = post2W[:, :f].T
    p2at = post2W[:, f:2 * f].T
    p2gt = post2W[:, 2 * f:].T
    l2t = lin2W.T

    b0r = b0[None, :]
    pre1br = pre1b[None, :]
    post1br = post1b[None, :]
    lin1br = lin1b[None, :]
    pre2br = pre2b[None, :]
    post2br = post2b[None, :]
    lin2br = lin2b[None, :]

    deg = jnp.zeros((n,), jnp.float32).at[dst].add(1.0)[:, None]

    h, a1, b1 = _dense_a(x, w0t, b0r, p1dt, p1st, pre1br)
    smax1 = _segmax_deg(b1, src, dst, n)
    h2, a2, b2 = _dense_b(h, a1, smax1, deg, p1ht, p1at, p1gt, post1br,
                          l1t, lin1br, p2dt, p2st, pre2br)
    smax2 = _segmax_deg(b2, src, dst, n)
    return _dense_c(h2, a2, smax2, deg, p2ht, p2at, p2gt, post2br,
                    l2t, lin2br, nclass)
